# baseline (speedup 1.0000x reference)
import jax
import jax.numpy as jnp
from jax.experimental import pallas as pl
from jax.experimental.pallas import tpu as pltpu

_BM = 400


def _proj_kernel(seq_ref, w_ref, sfts_ref):
    sfts = jax.lax.dot_general(
        seq_ref[...], w_ref[...],
        (((1,), (1,)), ((), ())),
        preferred_element_type=jnp.float32)
    sfts_ref[...] = sfts.astype(jnp.bfloat16)


def _spmm_kernel(a_ref, bias_ref, sfts_ref, adj_ref, out_ref):
    acc = jax.lax.dot_general(
        adj_ref[...].astype(jnp.bfloat16), sfts_ref[...],
        (((1,), (0,)), ((), ())),
        preferred_element_type=jnp.float32)
    acc = acc + bias_ref[...]
    a = a_ref[0, 0]
    out_ref[...] = jnp.where(acc >= 0, acc, a * acc)


def kernel(seq, adj, W, bias, prelu_a):
    n, in_ft = seq.shape
    out_ft = W.shape[0]
    a2 = jnp.reshape(prelu_a, (1, 1))
    bias2 = jnp.reshape(bias, (1, out_ft))
    sfts = pl.pallas_call(
        _proj_kernel,
        out_shape=jax.ShapeDtypeStruct((n, out_ft), jnp.bfloat16),
    )(seq, W)
    return pl.pallas_call(
        _spmm_kernel,
        grid=(n // _BM,),
        in_specs=[
            pl.BlockSpec(memory_space=pltpu.SMEM),
            pl.BlockSpec((1, out_ft), lambda i: (0, 0)),
            pl.BlockSpec((n, out_ft), lambda i: (0, 0)),
            pl.BlockSpec((_BM, n), lambda i: (i, 0)),
        ],
        out_specs=pl.BlockSpec((_BM, out_ft), lambda i: (i, 0)),
        out_shape=jax.ShapeDtypeStruct((n, out_ft), jnp.float32),
        compiler_params=pltpu.CompilerParams(
            dimension_semantics=("parallel",),
        ),
    )(a2, bias2, sfts, adj)


# speedup vs baseline: 1.0236x; 1.0236x over previous
import jax
import jax.numpy as jnp
from jax.experimental import pallas as pl
from jax.experimental.pallas import tpu as pltpu

_BH = 200


def _gcn_block_kernel(a_ref, bias_ref, seq_ref, w_ref, adj_a_ref, adj_b_ref,
                      out_ref, sfts_ref):
    @pl.when(pl.program_id(0) == 0)
    def _():
        sfts = jax.lax.dot_general(
            seq_ref[...], w_ref[...],
            (((1,), (1,)), ((), ())),
            preferred_element_type=jnp.float32)
        sfts_ref[...] = sfts.astype(jnp.bfloat16)

    a = a_ref[0, 0]
    acc_a = jax.lax.dot_general(
        adj_a_ref[...].astype(jnp.bfloat16), sfts_ref[...],
        (((1,), (0,)), ((), ())),
        preferred_element_type=jnp.float32)
    acc_a = acc_a + bias_ref[...]
    out_ref[:_BH, :] = jnp.where(acc_a >= 0, acc_a, a * acc_a)
    acc_b = jax.lax.dot_general(
        adj_b_ref[...].astype(jnp.bfloat16), sfts_ref[...],
        (((1,), (0,)), ((), ())),
        preferred_element_type=jnp.float32)
    acc_b = acc_b + bias_ref[...]
    out_ref[_BH:, :] = jnp.where(acc_b >= 0, acc_b, a * acc_b)


def kernel(seq, adj, W, bias, prelu_a):
    n, in_ft = seq.shape
    out_ft = W.shape[0]
    a2 = jnp.reshape(prelu_a, (1, 1))
    bias2 = jnp.reshape(bias, (1, out_ft))
    return pl.pallas_call(
        _gcn_block_kernel,
        grid=(n // (2 * _BH),),
        in_specs=[
            pl.BlockSpec(memory_space=pltpu.SMEM),
            pl.BlockSpec((1, out_ft), lambda i: (0, 0)),
            pl.BlockSpec((n, in_ft), lambda i: (0, 0)),
            pl.BlockSpec((out_ft, in_ft), lambda i: (0, 0)),
            pl.BlockSpec((_BH, n), lambda i: (2 * i, 0)),
            pl.BlockSpec((_BH, n), lambda i: (2 * i + 1, 0)),
        ],
        out_specs=pl.BlockSpec((2 * _BH, out_ft), lambda i: (i, 0)),
        out_shape=jax.ShapeDtypeStruct((n, out_ft), jnp.float32),
        scratch_shapes=[pltpu.VMEM((n, out_ft), jnp.bfloat16)],
        compiler_params=pltpu.CompilerParams(
            dimension_semantics=("arbitrary",),
        ),
    )(a2, bias2, seq, W, adj, adj)


# speedup vs baseline: 1.0325x; 1.0087x over previous
import jax
import jax.numpy as jnp
from jax.experimental import pallas as pl
from jax.experimental.pallas import tpu as pltpu

_BM = 400


def _gcn_block_kernel(a_ref, bias_ref, seq_ref, w_ref, adj_ref, out_ref,
                      sfts_ref):
    @pl.when(pl.program_id(0) == 0)
    def _():
        sfts = jax.lax.dot_general(
            seq_ref[...].astype(jnp.bfloat16),
            w_ref[...].astype(jnp.bfloat16),
            (((1,), (1,)), ((), ())),
            preferred_element_type=jnp.float32)
        sfts_ref[...] = sfts.astype(jnp.bfloat16)

    acc = jax.lax.dot_general(
        adj_ref[...].astype(jnp.bfloat16), sfts_ref[...],
        (((1,), (0,)), ((), ())),
        preferred_element_type=jnp.float32)
    acc = acc + bias_ref[...]
    a = a_ref[0, 0]
    out_ref[...] = jnp.where(acc >= 0, acc, a * acc)


def kernel(seq, adj, W, bias, prelu_a):
    n, in_ft = seq.shape
    out_ft = W.shape[0]
    a2 = jnp.reshape(prelu_a, (1, 1))
    bias2 = jnp.reshape(bias, (1, out_ft))
    return pl.pallas_call(
        _gcn_block_kernel,
        grid=(n // _BM,),
        in_specs=[
            pl.BlockSpec(memory_space=pltpu.SMEM),
            pl.BlockSpec((1, out_ft), lambda i: (0, 0)),
            pl.BlockSpec((n, in_ft), lambda i: (0, 0)),
            pl.BlockSpec((out_ft, in_ft), lambda i: (0, 0)),
            pl.BlockSpec((_BM, n), lambda i: (i, 0)),
        ],
        out_specs=pl.BlockSpec((_BM, out_ft), lambda i: (i, 0)),
        out_shape=jax.ShapeDtypeStruct((n, out_ft), jnp.float32),
        scratch_shapes=[pltpu.VMEM((n, out_ft), jnp.bfloat16)],
        compiler_params=pltpu.CompilerParams(
            dimension_semantics=("arbitrary",),
        ),
    )(a2, bias2, seq, W, adj)


# speedup vs baseline: 1.0445x; 1.0116x over previous
import jax
import jax.numpy as jnp
from jax.experimental import pallas as pl
from jax.experimental.pallas import tpu as pltpu

_BM = 400


def _gcn_block_kernel(a_ref, bias_ref, seq_ref, w_ref, adj_ref, out_ref,
                      sfts_ref):
    @pl.when(pl.program_id(0) == 0)
    def _():
        sfts = jax.lax.dot_general(
            seq_ref[...], w_ref[...],
            (((1,), (1,)), ((), ())),
            preferred_element_type=jnp.float32)
        sfts_ref[...] = sfts.astype(jnp.bfloat16)

    acc = jax.lax.dot_general(
        adj_ref[...].astype(jnp.bfloat16), sfts_ref[...],
        (((1,), (0,)), ((), ())),
        preferred_element_type=jnp.float32)
    acc = acc + bias_ref[...]
    a = a_ref[0, 0]
    out_ref[...] = jnp.where(acc >= 0, acc, a * acc)


def kernel(seq, adj, W, bias, prelu_a):
    n, in_ft = seq.shape
    out_ft = W.shape[0]
    a2 = jnp.reshape(prelu_a, (1, 1))
    bias2 = jnp.reshape(bias, (1, out_ft))
    return pl.pallas_call(
        _gcn_block_kernel,
        grid=(n // _BM,),
        in_specs=[
            pl.BlockSpec(memory_space=pltpu.SMEM),
            pl.BlockSpec((1, out_ft), lambda i: (0, 0)),
            pl.BlockSpec((n, in_ft), lambda i: (0, 0)),
            pl.BlockSpec((out_ft, in_ft), lambda i: (0, 0)),
            pl.BlockSpec((_BM, n), lambda i: (i, 0)),
        ],
        out_specs=pl.BlockSpec((_BM, out_ft), lambda i: (i, 0)),
        out_shape=jax.ShapeDtypeStruct((n, out_ft), jnp.float32),
        scratch_shapes=[pltpu.VMEM((n, out_ft), jnp.bfloat16)],
        compiler_params=pltpu.CompilerParams(
            dimension_semantics=("arbitrary",),
        ),
    )(a2, bias2, seq, W, adj)
